# Initial kernel scaffold; baseline (speedup 1.0000x reference)
import jax
import jax.numpy as jnp
from jax.experimental import pallas as pl


def kernel(x, edge_index, edge_attr, batch, W1, b1, W2, b2, Wl1, bl1, Wl2, bl2, Wl3, bl3):
    raise NotImplementedError("write your pallas kernel here")


# speedup vs baseline: 19.9334x; 19.9334x over previous
import functools

import jax
import jax.numpy as jnp
from jax import lax
from jax.experimental import pallas as pl
from jax.experimental.pallas import tpu as pltpu
from jax.experimental.pallas import tpu_sc as plsc

NN = 50000
NE = 800000
NG = 512
DIN = 32
DH = 128

N_PAD = 50176
E_PAD = 802816
NC = 2
NS = 16
EPT = E_PAD // NS
ROWS_PT = N_PAD // NS
STG = 56
N_OUTER = EPT // (STG * 128)
ERB = E_PAD // 128

_f32 = jnp.float32
_i32 = jnp.int32


def _mesh():
    return plsc.VectorSubcoreMesh(core_axis_name="c", subcore_axis_name="s",
                                  num_cores=NC, num_subcores=NS)


def _fill_const(ref, n16, value):
    def body(i, _):
        ref[pl.ds(i * 16, 16)] = jnp.full((16,), value, _f32)
        return 0
    lax.fori_loop(0, n16, body, 0)


def _fill2d_zero(ref, nrows, ncols):
    def body(r, _):
        for cc in range(ncols // 16):
            ref[r, pl.ds(cc * 16, 16)] = jnp.zeros((16,), _f32)
        return 0
    lax.fori_loop(0, nrows, body, 0)


def _deg_body(dst2d, ew2d, deg1_out, deg2_out, idx_b, val_b, zrow, accum, sem):
    c = lax.axis_index("c")
    s = lax.axis_index("s")

    _fill_const(zrow, ROWS_PT // 16, 0.0)

    @pl.when(c == 0)
    def _():
        def body(i, _):
            r = i // 8
            k = i % 8
            val_b[r, pl.ds(k * 16, 16)] = jnp.full((16,), 1.0, _f32)
            return 0
        lax.fori_loop(0, STG * 8, body, 0)

    pltpu.sync_copy(zrow, accum.at[pl.ds(s * ROWS_PT, ROWS_PT)])
    plsc.subcore_barrier()

    base = s * (EPT // 128)

    def outer(o, _):
        row0 = base + o * STG
        pltpu.sync_copy(dst2d.at[pl.ds(row0, STG)], idx_b)

        @pl.when(c == 1)
        def _():
            pltpu.sync_copy(ew2d.at[pl.ds(row0, STG)], val_b)

        def grp(h, _):
            descs = []
            for i2 in range(14):
                j = h * 14 + i2
                descs.append(pltpu.async_copy(
                    val_b.at[j], accum.at[idx_b.at[j]], sem, add=True))
            for d in descs:
                d.wait()
            return 0

        lax.fori_loop(0, 4, grp, 0)
        return 0

    lax.fori_loop(0, N_OUTER, outer, 0)
    plsc.subcore_barrier()

    sl = pl.ds(s * ROWS_PT, ROWS_PT)
    pltpu.sync_copy(accum.at[sl], zrow)

    @pl.when(c == 0)
    def _():
        pltpu.sync_copy(zrow, deg1_out.at[sl])

    @pl.when(c == 1)
    def _():
        pltpu.sync_copy(zrow, deg2_out.at[sl])


def _deg_call(dst2d, ew2d):
    return pl.kernel(
        _deg_body,
        out_type=[jax.ShapeDtypeStruct((N_PAD,), _f32),
                  jax.ShapeDtypeStruct((N_PAD,), _f32)],
        mesh=_mesh(),
        compiler_params=pltpu.CompilerParams(use_tc_tiling_on_sc=False),
        scratch_types=[
            pltpu.VMEM((STG, 128), _i32),
            pltpu.VMEM((STG, 128), _f32),
            pltpu.VMEM((ROWS_PT,), _f32),
            pltpu.VMEM_SHARED((N_PAD,), _f32),
            pltpu.SemaphoreType.DMA,
        ],
    )(dst2d, ew2d)


def _make_prop(C, weighted, P):
    n_in = 2 * P

    def body(*refs):
        g_refs = refs[:n_in]
        src2d = refs[n_in]
        dst2d = refs[n_in + 1]
        k = n_in + 2
        if weighted:
            ew2d = refs[k]
            k += 1
        out_refs = refs[k:k + n_in]
        k += n_in
        srcb, dstb = refs[k], refs[k + 1]
        k += 2
        if weighted:
            ewb = refs[k]
            k += 1
        rb = refs[k:k + 4]
        zb = refs[k + 4]
        bounce = refs[k + 5]
        accum = refs[k + 6]
        gsems = refs[k + 7:k + 11]
        ssems = refs[k + 11:k + 15]

        c = lax.axis_index("c")
        s = lax.axis_index("s")
        base = s * (EPT // 128)

        _fill2d_zero(zb, 196, C)

        def run_pass(g, out):
            for t in range(16):
                pltpu.sync_copy(
                    zb, accum.at[pl.ds(s * ROWS_PT + t * 196, 196)])
            plsc.subcore_barrier()

            def outer(o, _):
                row0 = base + o * STG
                pltpu.sync_copy(src2d.at[pl.ds(row0, STG)], srcb)
                pltpu.sync_copy(dst2d.at[pl.ds(row0, STG)], dstb)
                if weighted:
                    pltpu.sync_copy(ew2d.at[pl.ds(row0, STG)], ewb)

                pltpu.async_copy(g.at[srcb.at[0]], rb[0], gsems[0])
                pltpu.async_copy(g.at[srcb.at[1]], rb[1], gsems[1])

                def inner(j4, _):
                    for u in range(4):
                        j = j4 * 4 + u
                        nslot = (u + 2) % 4
                        buf = rb[u]
                        pltpu.make_async_copy(
                            g.at[srcb.at[j]], buf, gsems[u]).wait()

                        @pl.when(j + 2 < STG)
                        def _():
                            @pl.when(j >= 2)
                            def _():
                                pltpu.make_async_copy(
                                    rb[nslot], accum.at[dstb.at[j]],
                                    ssems[nslot]).wait()
                            pltpu.async_copy(
                                g.at[srcb.at[j + 2]], rb[nslot],
                                gsems[nslot])

                        if weighted:
                            def scale(eb, _):
                                wv = ewb[j, pl.ds(eb * 16, 16)]
                                for l in range(16):
                                    e = eb * 16 + l
                                    w = wv[l]
                                    for cc in range(C // 16):
                                        slc = pl.ds(cc * 16, 16)
                                        buf[e, slc] = buf[e, slc] * w
                                return 0
                            lax.fori_loop(0, 8, scale, 0)
                        pltpu.async_copy(
                            buf, accum.at[dstb.at[j]], ssems[u], add=True)
                    return 0

                lax.fori_loop(0, STG // 4, inner, 0)
                for u in range(4):
                    pltpu.make_async_copy(
                        rb[u], accum.at[dstb.at[u]], ssems[u]).wait()
                return 0

            lax.fori_loop(0, N_OUTER, outer, 0)
            plsc.subcore_barrier()
            for t in range(8):
                sl = pl.ds(s * ROWS_PT + t * 392, 392)
                pltpu.sync_copy(accum.at[sl], bounce)
                pltpu.sync_copy(bounce, out.at[sl])

        for half in range(NC):
            @pl.when(c == half)
            def _():
                for p_i in range(P):
                    run_pass(g_refs[half * P + p_i], out_refs[half * P + p_i])

    scratch = [
        pltpu.VMEM((STG, 128), _i32),
        pltpu.VMEM((STG, 128), _i32),
    ]
    if weighted:
        scratch.append(pltpu.VMEM((STG, 128), _f32))
    scratch += [
        pltpu.VMEM((128, C), _f32),
        pltpu.VMEM((128, C), _f32),
        pltpu.VMEM((128, C), _f32),
        pltpu.VMEM((128, C), _f32),
        pltpu.VMEM((196, C), _f32),
        pltpu.VMEM((392, C), _f32),
        pltpu.VMEM_SHARED((N_PAD, C), _f32),
    ] + [pltpu.SemaphoreType.DMA] * 8

    def call(g_arrays, src2d, dst2d, ew2d=None):
        args = list(g_arrays) + [src2d, dst2d]
        if weighted:
            args.append(ew2d)
        return pl.kernel(
            body,
            out_type=[jax.ShapeDtypeStruct((N_PAD, C), _f32)] * n_in,
            mesh=_mesh(),
            compiler_params=pltpu.CompilerParams(use_tc_tiling_on_sc=False),
            scratch_types=scratch,
        )(*args)

    return call


_prop_conv1 = _make_prop(16, False, 1)
_prop_conv2 = _make_prop(16, True, 4)


def _pool_body(h2, counts, sums_out, maxs_out, cnt_v, off_v, rbuf,
               sums_v, maxs_v):
    c = lax.axis_index("c")
    s = lax.axis_index("s")
    w = c * NS + s

    pltpu.sync_copy(counts, cnt_v)

    def pref(i, run):
        ch = cnt_v[pl.ds(i * 16, 16)]
        inc = plsc.cumsum(ch)
        off_v[pl.ds(i * 16, 16)] = inc - ch + run
        return run + jnp.sum(ch)

    lax.fori_loop(0, NG // 16, pref, jnp.int32(0))

    off16 = off_v[pl.ds(w * 16, 16)]
    cnt16 = cnt_v[pl.ds(w * 16, 16)]
    for k in range(16):
        st = off16[k]
        cnt = cnt16[k]
        nch = (cnt + 63) // 64

        def chunk(i, carry):
            accs, accm = carry
            pltpu.sync_copy(h2.at[pl.ds(st + i * 64, 64)], rbuf)

            def row(r, carry2):
                a_s, a_m = carry2
                valid = (i * 64 + r) < cnt
                new_s = []
                new_m = []
                for cc in range(8):
                    v = rbuf[r, pl.ds(cc * 16, 16)]
                    vz = jnp.where(valid, v, jnp.zeros((16,), _f32))
                    vm = jnp.where(valid, v,
                                   jnp.full((16,), -jnp.inf, _f32))
                    new_s.append(a_s[cc] + vz)
                    new_m.append(jnp.maximum(a_m[cc], vm))
                return tuple(new_s), tuple(new_m)

            return lax.fori_loop(0, 64, row, (accs, accm))

        z16 = jnp.zeros((16,), _f32)
        m16 = jnp.full((16,), -jnp.inf, _f32)
        accs, accm = lax.fori_loop(
            0, nch, chunk, (tuple(z16 for _ in range(8)),
                            tuple(m16 for _ in range(8))))
        for cc in range(8):
            sums_v[k, pl.ds(cc * 16, 16)] = accs[cc]
            maxs_v[k, pl.ds(cc * 16, 16)] = accm[cc]

    pltpu.sync_copy(sums_v, sums_out.at[pl.ds(w * 16, 16)])
    pltpu.sync_copy(maxs_v, maxs_out.at[pl.ds(w * 16, 16)])


def _pool_call(h2, counts):
    return pl.kernel(
        _pool_body,
        out_type=[jax.ShapeDtypeStruct((NG, DH), _f32),
                  jax.ShapeDtypeStruct((NG, DH), _f32)],
        mesh=_mesh(),
        compiler_params=pltpu.CompilerParams(use_tc_tiling_on_sc=False,
                                             needs_layout_passes=False),
        scratch_types=[
            pltpu.VMEM((NG,), _i32),
            pltpu.VMEM((NG,), _i32),
            pltpu.VMEM((64, DH), _f32),
            pltpu.VMEM((16, DH), _f32),
            pltpu.VMEM((16, DH), _f32),
        ],
    )(h2, counts)


_BLK = 1024
_NBLK = N_PAD // _BLK


def _k2_body(deg1, deg2, x, dinv1_o, dinv2_o, g0a_o, g0b_o):
    d1 = lax.rsqrt(deg1[...] + 1.0)
    d2 = lax.rsqrt(deg2[...] + 1.0)
    dinv1_o[...] = d1
    dinv2_o[...] = d2
    g0 = x[...] * d1
    g0a_o[...] = g0[:, :16]
    g0b_o[...] = g0[:, 16:]


def _k2_call(deg1c, deg2c, x_p):
    col = pl.BlockSpec((_BLK, 1), lambda i: (i, 0))
    return pl.pallas_call(
        _k2_body,
        grid=(_NBLK,),
        in_specs=[col, col, pl.BlockSpec((_BLK, DIN), lambda i: (i, 0))],
        out_specs=[col, col,
                   pl.BlockSpec((_BLK, 16), lambda i: (i, 0)),
                   pl.BlockSpec((_BLK, 16), lambda i: (i, 0))],
        out_shape=[jax.ShapeDtypeStruct((N_PAD, 1), _f32),
                   jax.ShapeDtypeStruct((N_PAD, 1), _f32),
                   jax.ShapeDtypeStruct((N_PAD, 16), _f32),
                   jax.ShapeDtypeStruct((N_PAD, 16), _f32)],
    )(deg1c, deg2c, x_p)


def _k4_body(p0a, p0b, g0a, g0b, dinv1, dinv2, W1, b1, *outs):
    t = jnp.concatenate([p0a[...] + g0a[...], p0b[...] + g0b[...]], axis=1)
    t = t * dinv1[...]
    h = jax.nn.relu(jnp.dot(t, W1[...], preferred_element_type=_f32)
                    + b1[...])
    g1 = h * dinv2[...]
    for q in range(8):
        outs[q][...] = g1[:, q * 16:(q + 1) * 16]


def _k4_call(p0a, p0b, g0a, g0b, dinv1, dinv2, W1, b1):
    col = pl.BlockSpec((_BLK, 1), lambda i: (i, 0))
    half = pl.BlockSpec((_BLK, 16), lambda i: (i, 0))
    return pl.pallas_call(
        _k4_body,
        grid=(_NBLK,),
        in_specs=[half, half, half, half, col, col,
                  pl.BlockSpec((DIN, DH), lambda i: (0, 0)),
                  pl.BlockSpec((1, DH), lambda i: (0, 0))],
        out_specs=[pl.BlockSpec((_BLK, 16), lambda i: (i, 0))] * 8,
        out_shape=[jax.ShapeDtypeStruct((N_PAD, 16), _f32)] * 8,
    )(p0a, p0b, g0a, g0b, dinv1, dinv2, W1, b1)


def _k6_body(*refs):
    p1 = refs[:8]
    g1 = refs[8:16]
    dinv2, W2, b2, batch = refs[16:20]
    h2_o, counts_o = refs[20], refs[21]
    i = pl.program_id(0)
    parts = [p1[q][...] + g1[q][...] for q in range(8)]
    t = jnp.concatenate(parts, axis=1) * dinv2[...]
    h2 = jax.nn.relu(jnp.dot(t, W2[...], preferred_element_type=_f32)
                     + b2[...])
    h2_o[...] = h2
    iota = lax.broadcasted_iota(_i32, (_BLK, NG), 1)
    eq = (batch[...] == iota).astype(_i32)
    cnt = jnp.sum(eq, axis=0, keepdims=True)

    @pl.when(i == 0)
    def _():
        counts_o[...] = cnt

    @pl.when(i > 0)
    def _():
        counts_o[...] = counts_o[...] + cnt


def _k6_call(p1s, g1s, dinv2, W2, b2, batch_col):
    col = pl.BlockSpec((_BLK, 1), lambda i: (i, 0))
    q = pl.BlockSpec((_BLK, 16), lambda i: (i, 0))
    return pl.pallas_call(
        _k6_body,
        grid=(_NBLK,),
        in_specs=[q] * 16 + [col,
                  pl.BlockSpec((DH, DH), lambda i: (0, 0)),
                  pl.BlockSpec((1, DH), lambda i: (0, 0)),
                  col],
        out_specs=[pl.BlockSpec((_BLK, DH), lambda i: (i, 0)),
                   pl.BlockSpec((1, NG), lambda i: (0, 0))],
        out_shape=[jax.ShapeDtypeStruct((N_PAD, DH), _f32),
                   jax.ShapeDtypeStruct((1, NG), _i32)],
    )(*p1s, *g1s, dinv2, W2, b2, batch_col)


def _k8_body(sums, maxs, cnt, Wl1a, Wl1b, bl1, Wl2, bl2, wl3, bl3, out):
    cf = cnt[...]
    x1 = sums[...] / jnp.maximum(cf, 1.0)
    x2 = jnp.where(cf > 0.0, maxs[...], 0.0)
    z = jax.nn.relu(jnp.dot(x1, Wl1a[...], preferred_element_type=_f32)
                    + jnp.dot(x2, Wl1b[...], preferred_element_type=_f32)
                    + bl1[...])
    z = jax.nn.relu(jnp.dot(z, Wl2[...], preferred_element_type=_f32)
                    + bl2[...])
    out[...] = jnp.sum(z * wl3[...], axis=1, keepdims=True) + bl3[...]


def _k8_call(sums, maxs, cnt_col, Wl1a, Wl1b, bl1, Wl2, bl2, wl3, bl3):
    full = lambda shape: pl.BlockSpec(shape, lambda: (0, 0))
    return pl.pallas_call(
        _k8_body,
        in_specs=[full((NG, DH)), full((NG, DH)), full((NG, 1)),
                  full((DH, DH)), full((DH, DH)), full((1, DH)),
                  full((DH, 64)), full((1, 64)), full((1, 64)),
                  full((1, 1))],
        out_specs=full((NG, 1)),
        out_shape=jax.ShapeDtypeStruct((NG, 1), _f32),
    )(sums, maxs, cnt_col, Wl1a, Wl1b, bl1, Wl2, bl2, wl3, bl3)


def kernel(x, edge_index, edge_attr, batch,
           W1, b1, W2, b2, Wl1, bl1, Wl2, bl2, Wl3, bl3):
    pe = E_PAD - NE
    src2d = jnp.concatenate(
        [edge_index[0], jnp.full((pe,), NN, _i32)]).reshape(ERB, 128)
    dst2d = jnp.concatenate(
        [edge_index[1], jnp.full((pe,), NN, _i32)]).reshape(ERB, 128)
    ew2d = jnp.concatenate(
        [edge_attr, jnp.zeros((pe,), _f32)]).reshape(ERB, 128)
    x_p = jnp.pad(x, ((0, N_PAD - NN), (0, 0)))
    batch_col = jnp.pad(batch, (0, N_PAD - NN),
                        constant_values=NG).reshape(N_PAD, 1)

    deg1, deg2 = _deg_call(dst2d, ew2d)
    dinv1, dinv2, g0a, g0b = _k2_call(
        deg1.reshape(N_PAD, 1), deg2.reshape(N_PAD, 1), x_p)

    p0a, p0b = _prop_conv1([g0a, g0b], src2d, dst2d)

    g1s = _k4_call(p0a, p0b, g0a, g0b, dinv1, dinv2,
                   W1, b1.reshape(1, DH))

    p1s = _prop_conv2(g1s, src2d, dst2d, ew2d)

    h2, counts = _k6_call(p1s, g1s, dinv2, W2, b2.reshape(1, DH), batch_col)

    sums, maxs = _pool_call(h2, counts.reshape(NG))

    out = _k8_call(sums, maxs, counts.reshape(NG, 1).astype(_f32),
                   Wl1[:DH], Wl1[DH:], bl1.reshape(1, DH),
                   Wl2, bl2.reshape(1, 64), Wl3.reshape(1, 64),
                   bl3.reshape(1, 1))
    return out.reshape(NG)
